# baseline (speedup 1.0000x reference)
import functools
import math

import jax
import jax.numpy as jnp
from jax import lax
from jax.experimental import pallas as pl
from jax.experimental.pallas import tpu as pltpu
from jax.experimental.pallas import tpu_sc as plsc

D_MODEL = 128
SCALE = math.sqrt(float(D_MODEL))
LANES = 16


def _sc_embed_concat(xt5, lut, *, n_b, n_s, n_f):
    f_keep = n_f - 1
    d_out = f_keep + D_MODEL
    sh_n = n_s // 8
    bh_n = n_b // 128
    n_items = sh_n * bh_n

    info = plsc.get_sparse_core_info()
    nc, ns = info.num_cores, info.num_subcores
    nw = nc * ns

    mesh = plsc.VectorSubcoreMesh(core_axis_name="c", subcore_axis_name="s")

    @functools.partial(
        pl.kernel,
        mesh=mesh,
        out_type=jax.ShapeDtypeStruct((n_s, d_out // 8, bh_n, 8, 128), jnp.float32),
        scratch_types=[
            pltpu.VMEM((n_f, 8, 128), jnp.float32),
            pltpu.VMEM((8, 128), jnp.int32),
            pltpu.VMEM((128, D_MODEL), jnp.float32),
            pltpu.VMEM((d_out // 8, 8, 128), jnp.float32),
            pltpu.SemaphoreType.DMA,
        ],
        compiler_params=pltpu.CompilerParams(needs_layout_passes=False),
    )
    def k(xt_hbm, lut_hbm, out_hbm, x_st, idx_v, emb_v, out_st, sem):
        wid = lax.axis_index("s") * nc + lax.axis_index("c")
        n_mine = (n_items - 1 - wid) // nw + 1

        row_ids = [jnp.arange(LANES, dtype=jnp.int32) + g * LANES for g in range(8)]

        def item_body(it, carry):
            ii = wid + it * nw
            sh = ii // bh_n
            bh = ii % bh_n

            pltpu.sync_copy(xt_hbm.at[:, sh, bh], x_st)

            for g in range(8):
                for l in range(8):
                    sl = pl.ds(l * LANES, LANES)
                    idx_v[g, sl] = x_st[f_keep, g, sl].astype(jnp.int32)

            for sl_i in range(8):
                pltpu.async_copy(lut_hbm.at[idx_v.at[sl_i]], emb_v, sem).wait()

                for c in range(f_keep):
                    for g in range(8):
                        lsl = pl.ds(g * LANES, LANES)
                        out_st[c // 8, c % 8, lsl] = x_st[c, sl_i, lsl]

                def ch_body(c, c2):
                    cc = c + f_keep
                    ch = lax.shift_right_logical(cc, 3)
                    cl = lax.bitwise_and(cc, 7)
                    col = jnp.full((LANES,), c, dtype=jnp.int32)
                    for g in range(8):
                        v = plsc.load_gather(emb_v, [row_ids[g], col]) * SCALE
                        out_st[ch, cl, pl.ds(g * LANES, LANES)] = v
                    return c2

                lax.fori_loop(0, D_MODEL, ch_body, 0)

                pltpu.sync_copy(out_st, out_hbm.at[sh * 8 + sl_i, :, bh])

            return carry

        lax.fori_loop(0, n_mine, item_body, 0)

    return k(xt5, lut)


def kernel(x, lut):
    b, s, f = x.shape
    xt5 = (
        x.transpose(2, 1, 0)
        .reshape(f, s // 8, 8, b // 128, 128)
        .transpose(0, 1, 3, 2, 4)
    )
    out5 = _sc_embed_concat(xt5, lut, n_b=b, n_s=s, n_f=f)
    d_out = f - 1 + D_MODEL
    out = (
        out5.transpose(0, 1, 3, 2, 4)
        .reshape(s, d_out, b)
        .transpose(2, 0, 1)
    )
    return out


# speedup vs baseline: 1.6390x; 1.6390x over previous
import functools
import math

import jax
import jax.numpy as jnp
from jax import lax
from jax.experimental import pallas as pl
from jax.experimental.pallas import tpu as pltpu
from jax.experimental.pallas import tpu_sc as plsc

D_MODEL = 128
SCALE = math.sqrt(float(D_MODEL))
LANES = 16
PAD = 129


def _sc_embed_concat(xt5, lut, *, n_b, n_s, n_f):
    f_keep = n_f - 1
    d_out = f_keep + D_MODEL
    sh_n = n_s // 8
    bh_n = n_b // 128
    n_items = sh_n * bh_n

    info = plsc.get_sparse_core_info()
    nc, ns = info.num_cores, info.num_subcores
    nw = nc * ns

    mesh = plsc.VectorSubcoreMesh(core_axis_name="c", subcore_axis_name="s")

    @functools.partial(
        pl.kernel,
        mesh=mesh,
        out_type=jax.ShapeDtypeStruct((n_s, d_out // 8, bh_n, 8, 128), jnp.float32),
        scratch_types=[
            pltpu.VMEM((n_f, 8, 128), jnp.float32),
            pltpu.VMEM((8, 128), jnp.int32),
            pltpu.VMEM((128, D_MODEL), jnp.float32),
            pltpu.VMEM((d_out // 8, 8, PAD), jnp.float32),
            pltpu.SemaphoreType.DMA,
        ],
        compiler_params=pltpu.CompilerParams(needs_layout_passes=False),
    )
    def k(xt_hbm, lut_hbm, out_hbm, x_st, idx_v, emb_v, out_pad, sem):
        wid = lax.axis_index("s") * nc + lax.axis_index("c")
        n_mine = (n_items - 1 - wid) // nw + 1

        lane = jnp.arange(LANES, dtype=jnp.int32)
        d01 = []
        for j in range(D_MODEL // LANES):
            cc = f_keep + j * LANES + lane
            d01.append((cc >> 3, cc & 7))

        def item_body(it, carry):
            ii = wid + it * nw
            sh = ii // bh_n
            bh = ii % bh_n

            pltpu.sync_copy(xt_hbm.at[:, sh, bh], x_st)

            for g in range(8):
                for l in range(8):
                    sl = pl.ds(l * LANES, LANES)
                    idx_v[g, sl] = x_st[f_keep, g, sl].astype(jnp.int32)

            for sl_i in range(8):
                pltpu.async_copy(lut_hbm.at[idx_v.at[sl_i]], emb_v, sem).wait()

                for c in range(f_keep):
                    for g in range(8):
                        out_pad[c // 8, c % 8, pl.ds(g * LANES, LANES)] = (
                            x_st[c, sl_i, pl.ds(g * LANES, LANES)]
                        )

                @plsc.parallel_loop(0, 128, unroll=2)
                def _(t):
                    d2 = jnp.full((LANES,), t, dtype=jnp.int32)
                    for j in range(D_MODEL // LANES):
                        v = emb_v[t, pl.ds(j * LANES, LANES)] * SCALE
                        plsc.store_scatter(out_pad, [d01[j][0], d01[j][1], d2], v)

                pltpu.sync_copy(
                    out_pad.at[:, :, pl.ds(0, 128)],
                    out_hbm.at[sh * 8 + sl_i, :, bh],
                )

            return carry

        lax.fori_loop(0, n_mine, item_body, 0)

    return k(xt5, lut)


def kernel(x, lut):
    b, s, f = x.shape
    xt5 = (
        x.transpose(2, 1, 0)
        .reshape(f, s // 8, 8, b // 128, 128)
        .transpose(0, 1, 3, 2, 4)
    )
    out5 = _sc_embed_concat(xt5, lut, n_b=b, n_s=s, n_f=f)
    d_out = f - 1 + D_MODEL
    out = (
        out5.transpose(0, 1, 3, 2, 4)
        .reshape(s, d_out, b)
        .transpose(2, 0, 1)
    )
    return out


# speedup vs baseline: 2.0321x; 1.2398x over previous
import functools
import math

import jax
import jax.numpy as jnp
from jax import lax
from jax.experimental import pallas as pl
from jax.experimental.pallas import tpu as pltpu
from jax.experimental.pallas import tpu_sc as plsc

D_MODEL = 128
SCALE = math.sqrt(float(D_MODEL))
LANES = 16
PAD = 129


def _sc_embed_concat(xt5, lut, *, n_b, n_s, n_f):
    f_keep = n_f - 1
    d_out = f_keep + D_MODEL
    sh_n = n_s // 8
    bh_n = n_b // 128
    n_items = sh_n * bh_n

    info = plsc.get_sparse_core_info()
    nc, ns = info.num_cores, info.num_subcores
    nw = nc * ns

    mesh = plsc.VectorSubcoreMesh(core_axis_name="c", subcore_axis_name="s")

    @functools.partial(
        pl.kernel,
        mesh=mesh,
        out_type=jax.ShapeDtypeStruct((n_s, d_out // 8, bh_n, 8, 128), jnp.float32),
        scratch_types=[
            pltpu.VMEM((n_f, 8, 128), jnp.float32),
            pltpu.VMEM((8, 128), jnp.int32),
            pltpu.VMEM((2, 128, D_MODEL), jnp.float32),
            pltpu.VMEM((2, d_out // 8, 8, PAD), jnp.float32),
            pltpu.SemaphoreType.DMA,
            pltpu.SemaphoreType.DMA,
            pltpu.SemaphoreType.DMA,
            pltpu.SemaphoreType.DMA,
        ],
        compiler_params=pltpu.CompilerParams(needs_layout_passes=False),
    )
    def k(xt_hbm, lut_hbm, out_hbm, x_st, idx_v, emb_v, out_pad,
          gsem0, gsem1, osem0, osem1):
        gsem = (gsem0, gsem1)
        osem = (osem0, osem1)
        wid = lax.axis_index("s") * nc + lax.axis_index("c")
        n_mine = (n_items - 1 - wid) // nw + 1

        lane = jnp.arange(LANES, dtype=jnp.int32)
        d01 = []
        for j in range(D_MODEL // LANES):
            cc = f_keep + j * LANES + lane
            d01.append((cc >> 3, cc & 7))

        def item_body(it, carry):
            ii = wid + it * nw
            sh = ii // bh_n
            bh = ii % bh_n

            pltpu.sync_copy(xt_hbm.at[:, sh, bh], x_st)

            for g in range(8):
                for l in range(8):
                    sl = pl.ds(l * LANES, LANES)
                    idx_v[g, sl] = x_st[f_keep, g, sl].astype(jnp.int32)

            gcp = [None, None]
            ocp = [None, None]
            gcp[0] = pltpu.async_copy(
                lut_hbm.at[idx_v.at[0]], emb_v.at[0], gsem[0]
            )
            for sl_i in range(8):
                buf = sl_i % 2
                gcp[buf].wait()
                if sl_i < 7:
                    gcp[1 - buf] = pltpu.async_copy(
                        lut_hbm.at[idx_v.at[sl_i + 1]],
                        emb_v.at[1 - buf],
                        gsem[1 - buf],
                    )
                if ocp[buf] is not None:
                    ocp[buf].wait()

                for c in range(f_keep):
                    for g in range(8):
                        out_pad[buf, c // 8, c % 8, pl.ds(g * LANES, LANES)] = (
                            x_st[c, sl_i, pl.ds(g * LANES, LANES)]
                        )

                emb_b = emb_v.at[buf]
                out_b = out_pad.at[buf]

                @plsc.parallel_loop(0, 128, unroll=2)
                def _(t):
                    d2 = jnp.full((LANES,), t, dtype=jnp.int32)
                    for j in range(D_MODEL // LANES):
                        v = emb_b[t, pl.ds(j * LANES, LANES)] * SCALE
                        plsc.store_scatter(out_b, [d01[j][0], d01[j][1], d2], v)

                ocp[buf] = pltpu.async_copy(
                    out_pad.at[buf, :, :, pl.ds(0, 128)],
                    out_hbm.at[sh * 8 + sl_i, :, bh],
                    osem[buf],
                )

            ocp[0].wait()
            ocp[1].wait()
            return carry

        lax.fori_loop(0, n_mine, item_body, 0)

    return k(xt5, lut)


def kernel(x, lut):
    b, s, f = x.shape
    xt5 = (
        x.transpose(2, 1, 0)
        .reshape(f, s // 8, 8, b // 128, 128)
        .transpose(0, 1, 3, 2, 4)
    )
    out5 = _sc_embed_concat(xt5, lut, n_b=b, n_s=s, n_f=f)
    d_out = f - 1 + D_MODEL
    out = (
        out5.transpose(0, 1, 3, 2, 4)
        .reshape(s, d_out, b)
        .transpose(2, 0, 1)
    )
    return out
